# baseline (speedup 1.0000x reference)
import jax
import jax.numpy as jnp
from jax.experimental import pallas as pl
from jax.experimental.pallas import tpu as pltpu

NUM_EXPERTS = 64
TOP_K = 8
T_BLK = 256
NBUF = 4


def _gating_kernel(x_hbm, w_ref, b_ref, sw_ref, idx_ref, buf, acc_ref, sem):
    B, T, D = x_hbm.shape
    n_steps = T // T_BLK

    def copy(step, slot):
        return pltpu.make_async_copy(
            x_hbm.at[:, pl.ds(step * T_BLK, T_BLK), :],
            buf.at[slot],
            sem.at[slot],
        )

    for s in range(min(NBUF, n_steps)):
        copy(s, s).start()

    acc_ref[...] = jnp.zeros_like(acc_ref)
    for step in range(n_steps):
        slot = step % NBUF
        copy(step, slot).wait()
        acc_ref[...] += jnp.sum(buf[slot], axis=1)
        nxt = step + NBUF
        if nxt < n_steps:
            copy(nxt, slot).start()

    pooled = acc_ref[...] * (1.0 / T)
    logits = jax.lax.dot_general(
        pooled, w_ref[...], (((1,), (1,)), ((), ())),
        precision=jax.lax.Precision.HIGHEST,
        preferred_element_type=jnp.float32,
    ) + b_ref[...]

    e_iota = jax.lax.broadcasted_iota(jnp.int32, logits.shape, 1)
    vals = logits
    top_vals = []
    top_idx = []
    for _ in range(TOP_K):
        m = jnp.max(vals, axis=1, keepdims=True)
        i = jnp.min(jnp.where(vals == m, e_iota, NUM_EXPERTS),
                    axis=1, keepdims=True)
        top_vals.append(m)
        top_idx.append(i)
        vals = jnp.where(e_iota == i, -jnp.inf, vals)

    tv = jnp.concatenate(top_vals, axis=1)
    ex = jnp.exp(tv - tv[:, :1])
    probs = ex / jnp.sum(ex, axis=1, keepdims=True)

    sparse = jnp.zeros_like(logits)
    for k in range(TOP_K):
        sparse += jnp.where(e_iota == top_idx[k], probs[:, k:k + 1], 0.0)

    sw_ref[...] = sparse
    idx_ref[...] = jnp.concatenate(top_idx, axis=1)


@jax.jit
def kernel(x, W, b):
    B, T, D = x.shape
    sw, idx = pl.pallas_call(
        _gating_kernel,
        in_specs=[
            pl.BlockSpec(memory_space=pl.ANY),
            pl.BlockSpec((NUM_EXPERTS, D), lambda: (0, 0)),
            pl.BlockSpec((1, NUM_EXPERTS), lambda: (0, 0)),
        ],
        out_specs=[
            pl.BlockSpec((B, NUM_EXPERTS), lambda: (0, 0)),
            pl.BlockSpec((B, TOP_K), lambda: (0, 0)),
        ],
        out_shape=[
            jax.ShapeDtypeStruct((B, NUM_EXPERTS), jnp.float32),
            jax.ShapeDtypeStruct((B, TOP_K), jnp.int32),
        ],
        scratch_shapes=[
            pltpu.VMEM((NBUF, B, T_BLK, D), jnp.float32),
            pltpu.VMEM((B, D), jnp.float32),
            pltpu.SemaphoreType.DMA((NBUF,)),
        ],
    )(x, W, b.reshape(1, NUM_EXPERTS))
    return (sw, idx)
